# baseline (speedup 1.0000x reference)
import jax
import jax.numpy as jnp
from jax import lax
from jax.experimental import pallas as pl
from jax.experimental.pallas import tpu as pltpu
from jax.experimental.pallas import tpu_sc as plsc

VOCAB = 1000000
EMB = 32
BATCH = 4096
HIST = 200

NC = 2
NS = 16
NW = NC * NS

BW = BATCH // NW
LANES = 16
NG = 2
TROWS = VOCAB // 4

TCOLS = VOCAB // 128
CB = 4
NCB = TCOLS // CB
TREM = VOCAB - TCOLS * 128


def _transpose_block(src, dst, jmax):
    lane = lax.iota(jnp.int32, LANES)
    lane32 = lane * EMB

    def diag4(d4, carry):
        for dd in range(4):
            d = d4 * 4 + dd
            emod = jnp.bitwise_and(lane + d, LANES - 1)
            base = lane32 + emod
            for eg in (0, LANES):
                for j0 in range(0, jmax, LANES):
                    jv = lane + j0
                    v = plsc.load_gather(src, [emod + eg, jv])
                    addr = base + (j0 * EMB + eg)
                    plsc.store_scatter(
                        dst, [lax.shift_right_logical(addr, 7),
                              jnp.bitwise_and(addr, 127)], v)
        return carry

    lax.fori_loop(0, LANES // 4, diag4, 0)


def _relayout_body(tbl_t, out_hbm, ibufs, obufs, isems, osems):
    wid = lax.axis_index("s") * NC + lax.axis_index("c")
    nblk = (NCB + NW - 1) // NW

    def fire_read(i, p):
        c0 = pl.multiple_of((wid + NW * i) * (CB * 128), 128)
        pltpu.async_copy(tbl_t.at[:, pl.ds(c0, CB * 128)], ibufs[p],
                         isems[p])

    def wait_read(p):
        pltpu.make_async_copy(tbl_t.at[:, pl.ds(0, CB * 128)], ibufs[p],
                              isems[p]).wait()

    def fire_write(i, p):
        r0 = pl.multiple_of((wid + NW * i) * (CB * EMB), EMB)
        pltpu.async_copy(obufs[p], out_hbm.at[pl.ds(r0, CB * EMB)], osems[p])

    def wait_write(p):
        pltpu.make_async_copy(obufs[p], out_hbm.at[pl.ds(0, CB * EMB)],
                              osems[p]).wait()

    fire_read(0, 0)

    def step(i2, carry):
        for k in range(2):
            i = 2 * i2 + k
            p = k

            @pl.when(wid + NW * i < NCB)
            def _():
                @pl.when(wid + NW * (i + 1) < NCB)
                def _():
                    fire_read(i + 1, 1 - p)

                wait_read(p)

                @pl.when(i >= 2)
                def _():
                    wait_write(p)

                _transpose_block(ibufs[p], obufs[p], CB * 128)
                fire_write(i, p)

        return carry

    lax.fori_loop(0, (nblk + 1) // 2, step, 0)
    wait_write(0)
    wait_write(1)

    @pl.when(wid == 0)
    def _():
        cstart = pl.multiple_of(jnp.int32(TCOLS) * 128, 128)
        pltpu.async_copy(tbl_t.at[:, pl.ds(cstart, 128)],
                         ibufs[0].at[:, pl.ds(0, 128)], isems[0])
        pltpu.make_async_copy(tbl_t.at[:, pl.ds(0, 128)],
                              ibufs[0].at[:, pl.ds(0, 128)], isems[0]).wait()
        _transpose_block(ibufs[0], obufs[0], TREM)
        nrow = (TREM * EMB) // 128
        pltpu.sync_copy(obufs[0].at[pl.ds(0, nrow)],
                        out_hbm.at[pl.ds(TCOLS * EMB, nrow)])


def _gather_body(table_hbm, idx_hbm, out_hbm, idx_v, gbufs, tbufs, rbufs,
                 gsems, wsems):
    wid = lax.axis_index("s") * NC + lax.axis_index("c")
    b0 = wid * BW
    pltpu.sync_copy(idx_hbm.at[:, pl.ds(b0, BW)], idx_v)

    def fire_gather(h, g):
        r = rbufs[g]
        for j0 in range(0, BW, LANES):
            v = idx_v[h, pl.ds(j0, LANES)]
            r[pl.ds(j0, LANES)] = lax.shift_right_logical(v, 2)
        pltpu.async_copy(table_hbm.at[r], gbufs[g], gsems[g])

    def wait_gather(g):
        pltpu.make_async_copy(table_hbm.at[rbufs[0]], gbufs[g],
                              gsems[g]).wait()

    def fire_write(h, p):
        pltpu.async_copy(tbufs[p], out_hbm.at[h, :, pl.ds(b0, BW)], wsems[p])

    def wait_write(p):
        pltpu.make_async_copy(tbufs[p], out_hbm.at[0, :, pl.ds(b0, BW)],
                              wsems[p]).wait()

    def transpose(h, g, p):
        src, dst = gbufs[g], tbufs[p]
        lane = lax.iota(jnp.int32, LANES)
        qs = [lax.shift_left(jnp.bitwise_and(idx_v[h, pl.ds(j0, LANES)], 3),
                             5)
              for j0 in range(0, BW, LANES)]
        rows = [lane + j0 for j0 in range(0, BW, LANES)]

        def diag(d, carry):
            emod = jnp.bitwise_and(lane + d, LANES - 1)
            for eg in (0, LANES):
                erow = emod + eg
                for i in range(BW // LANES):
                    v = plsc.load_gather(src, [rows[i], qs[i] + erow])
                    plsc.store_scatter(dst, [erow, rows[i]], v)
            return carry

        lax.fori_loop(0, LANES, diag, 0)

    for g in range(NG):
        fire_gather(g, g)

    def step(i, carry):
        h = NG * i
        for k in range(NG):
            p = k % 2
            wait_gather(k)

            @pl.when(i > 0)
            def _():
                wait_write(p)

            transpose(h + k, k, p)

            @pl.when(h + k + NG < HIST)
            def _():
                fire_gather(h + k + NG, k)

            fire_write(h + k, p)
        return carry

    lax.fori_loop(0, HIST // NG, step, 0)
    wait_write(0)
    wait_write(1)


@jax.jit
def _run(token_table_t, idx_t):
    mesh = plsc.VectorSubcoreMesh(core_axis_name="c", subcore_axis_name="s")
    relayout = pl.kernel(
        _relayout_body,
        out_type=jax.ShapeDtypeStruct((TROWS, 128), jnp.float32),
        mesh=mesh,
        scratch_types=[
            [pltpu.VMEM((EMB, CB * 128), jnp.float32) for _ in range(2)],
            [pltpu.VMEM((CB * EMB, 128), jnp.float32) for _ in range(2)],
            [pltpu.SemaphoreType.DMA for _ in range(2)],
            [pltpu.SemaphoreType.DMA for _ in range(2)],
        ],
        compiler_params=pltpu.CompilerParams(use_tc_tiling_on_sc=True,
                                             needs_layout_passes=False,
                                             disable_bounds_checks=True),
    )
    table128 = relayout(token_table_t)

    gather = pl.kernel(
        _gather_body,
        out_type=jax.ShapeDtypeStruct((HIST, EMB, BATCH), jnp.float32),
        mesh=mesh,
        scratch_types=[
            pltpu.VMEM((HIST, BW), jnp.int32),
            [pltpu.VMEM((BW, 128), jnp.float32) for _ in range(NG)],
            [pltpu.VMEM((EMB, BW), jnp.float32) for _ in range(2)],
            [pltpu.VMEM((BW,), jnp.int32) for _ in range(NG)],
            [pltpu.SemaphoreType.DMA for _ in range(NG)],
            [pltpu.SemaphoreType.DMA for _ in range(2)],
        ],
        compiler_params=pltpu.CompilerParams(use_tc_tiling_on_sc=True,
                                             needs_layout_passes=False),
    )
    return gather(table128, idx_t)


def kernel(node_feats, node_lens, token_table):
    del node_lens
    idx_t = node_feats.T.astype(jnp.int32)
    out_t = _run(token_table.T, idx_t)
    return out_t.transpose(2, 0, 1)


# speedup vs baseline: 1.0292x; 1.0292x over previous
import jax
import jax.numpy as jnp
from jax import lax
from jax.experimental import pallas as pl
from jax.experimental.pallas import tpu as pltpu
from jax.experimental.pallas import tpu_sc as plsc

VOCAB = 1000000
EMB = 32
BATCH = 4096
HIST = 200

NC = 2
NS = 16
NW = NC * NS

BW = BATCH // NW
LANES = 16
NG = 2
TROWS = VOCAB // 4

TCOLS = VOCAB // 128
TREM = VOCAB - TCOLS * 128


def _transpose_block(src, dst, jmax):
    lane = lax.iota(jnp.int32, LANES)
    lane32 = lane * EMB

    def diag4(d4, carry):
        for dd in range(4):
            d = d4 * 4 + dd
            emod = jnp.bitwise_and(lane + d, LANES - 1)
            base = lane32 + emod
            for eg in (0, LANES):
                for j0 in range(0, jmax, LANES):
                    jv = lane + j0
                    v = plsc.load_gather(src, [emod + eg, jv])
                    addr = base + (j0 * EMB + eg)
                    plsc.store_scatter(
                        dst, [lax.shift_right_logical(addr, 7),
                              jnp.bitwise_and(addr, 127)], v)
        return carry

    lax.fori_loop(0, LANES // 4, diag4, 0)


def _relayout_body(tbl_t, out_hbm, ibufs, obufs, isems, osems):
    wid = lax.axis_index("s") * NC + lax.axis_index("c")
    nblk = (TCOLS + NW - 1) // NW

    def fire_read(i, p):
        c0 = pl.multiple_of((wid + NW * i) * 128, 128)
        pltpu.async_copy(tbl_t.at[:, pl.ds(c0, 128)], ibufs[p], isems[p])

    def wait_read(p):
        pltpu.make_async_copy(tbl_t.at[:, pl.ds(0, 128)], ibufs[p],
                              isems[p]).wait()

    def fire_write(i, p):
        r0 = pl.multiple_of((wid + NW * i) * EMB, EMB)
        pltpu.async_copy(obufs[p], out_hbm.at[pl.ds(r0, EMB)], osems[p])

    def wait_write(p):
        pltpu.make_async_copy(obufs[p], out_hbm.at[pl.ds(0, EMB)],
                              osems[p]).wait()

    fire_read(0, 0)

    def step(i2, carry):
        for k in range(2):
            i = 2 * i2 + k
            p = k

            @pl.when(wid + NW * i < TCOLS)
            def _():
                @pl.when(wid + NW * (i + 1) < TCOLS)
                def _():
                    fire_read(i + 1, 1 - p)

                wait_read(p)

                @pl.when(i >= 2)
                def _():
                    wait_write(p)

                _transpose_block(ibufs[p], obufs[p], 128)
                fire_write(i, p)

        return carry

    lax.fori_loop(0, (nblk + 1) // 2, step, 0)
    wait_write(0)
    wait_write(1)

    @pl.when(wid == 0)
    def _():
        cstart = pl.multiple_of(jnp.int32(TCOLS) * 128, 128)
        pltpu.async_copy(tbl_t.at[:, pl.ds(cstart, 128)], ibufs[0], isems[0])
        pltpu.make_async_copy(tbl_t.at[:, pl.ds(0, 128)], ibufs[0],
                              isems[0]).wait()
        _transpose_block(ibufs[0], obufs[0], TREM)
        nrow = (TREM * EMB) // 128
        pltpu.sync_copy(obufs[0].at[pl.ds(0, nrow)],
                        out_hbm.at[pl.ds(TCOLS * EMB, nrow)])


def _gather_body(table_hbm, idx_hbm, out_hbm, idx_v, gbufs, tbufs, rbufs,
                 gsems, wsems):
    wid = lax.axis_index("s") * NC + lax.axis_index("c")
    b0 = wid * BW
    pltpu.sync_copy(idx_hbm.at[:, pl.ds(b0, BW)], idx_v)

    def fire_gather(h, g):
        r = rbufs[g]
        for j0 in range(0, BW, LANES):
            v = idx_v[h, pl.ds(j0, LANES)]
            r[pl.ds(j0, LANES)] = lax.shift_right_logical(v, 2)
        pltpu.async_copy(table_hbm.at[r], gbufs[g], gsems[g])

    def wait_gather(g):
        pltpu.make_async_copy(table_hbm.at[rbufs[0]], gbufs[g],
                              gsems[g]).wait()

    def fire_write(h, p):
        pltpu.async_copy(tbufs[p], out_hbm.at[h, :, pl.ds(b0, BW)], wsems[p])

    def wait_write(p):
        pltpu.make_async_copy(tbufs[p], out_hbm.at[0, :, pl.ds(b0, BW)],
                              wsems[p]).wait()

    def transpose(h, g, p):
        src, dst = gbufs[g], tbufs[p]
        lane = lax.iota(jnp.int32, LANES)
        qs = [lax.shift_left(jnp.bitwise_and(idx_v[h, pl.ds(j0, LANES)], 3),
                             5)
              for j0 in range(0, BW, LANES)]
        rows = [lane + j0 for j0 in range(0, BW, LANES)]

        def diag(d, carry):
            emod = jnp.bitwise_and(lane + d, LANES - 1)
            for eg in (0, LANES):
                erow = emod + eg
                for i in range(BW // LANES):
                    v = plsc.load_gather(src, [rows[i], qs[i] + erow])
                    plsc.store_scatter(dst, [erow, rows[i]], v)
            return carry

        lax.fori_loop(0, LANES, diag, 0)

    for g in range(NG):
        fire_gather(g, g)

    def step(i, carry):
        h = NG * i
        for k in range(NG):
            p = k % 2
            wait_gather(k)

            @pl.when(i > 0)
            def _():
                wait_write(p)

            transpose(h + k, k, p)

            @pl.when(h + k + NG < HIST)
            def _():
                fire_gather(h + k + NG, k)

            fire_write(h + k, p)
        return carry

    lax.fori_loop(0, HIST // NG, step, 0)
    wait_write(0)
    wait_write(1)


@jax.jit
def _run(token_table_t, idx_t):
    mesh = plsc.VectorSubcoreMesh(core_axis_name="c", subcore_axis_name="s")
    relayout = pl.kernel(
        _relayout_body,
        out_type=jax.ShapeDtypeStruct((TROWS, 128), jnp.float32),
        mesh=mesh,
        scratch_types=[
            [pltpu.VMEM((EMB, 128), jnp.float32) for _ in range(2)],
            [pltpu.VMEM((EMB, 128), jnp.float32) for _ in range(2)],
            [pltpu.SemaphoreType.DMA for _ in range(2)],
            [pltpu.SemaphoreType.DMA for _ in range(2)],
        ],
        compiler_params=pltpu.CompilerParams(use_tc_tiling_on_sc=True,
                                             needs_layout_passes=False,
                                             disable_bounds_checks=True),
    )
    table128 = relayout(token_table_t)

    gather = pl.kernel(
        _gather_body,
        out_type=jax.ShapeDtypeStruct((HIST, EMB, BATCH), jnp.float32),
        mesh=mesh,
        scratch_types=[
            pltpu.VMEM((HIST, BW), jnp.int32),
            [pltpu.VMEM((BW, 128), jnp.float32) for _ in range(NG)],
            [pltpu.VMEM((EMB, BW), jnp.float32) for _ in range(2)],
            [pltpu.VMEM((BW,), jnp.int32) for _ in range(NG)],
            [pltpu.SemaphoreType.DMA for _ in range(NG)],
            [pltpu.SemaphoreType.DMA for _ in range(2)],
        ],
        compiler_params=pltpu.CompilerParams(use_tc_tiling_on_sc=True,
                                             needs_layout_passes=False),
    )
    return gather(table128, idx_t)


def kernel(node_feats, node_lens, token_table):
    del node_lens
    idx_t = node_feats.T.astype(jnp.int32)
    out_t = _run(token_table.T, idx_t)
    return out_t.transpose(2, 0, 1)
